# baseline (speedup 1.0000x reference)
import functools

import jax
import jax.numpy as jnp
from jax import lax
from jax.experimental import pallas as pl
from jax.experimental.pallas import tpu as pltpu
from jax.experimental.pallas import tpu_sc as plsc

EMBED = 32
BATCH = 16384

NUM_CORES = 2
NUM_SUBCORES = 16
NUM_WORKERS = NUM_CORES * NUM_SUBCORES
B_PER_W = BATCH // NUM_WORKERS
CHUNK = 128
N_CHUNKS = B_PER_W // CHUNK


@functools.partial(
    pl.kernel,
    mesh=plsc.VectorSubcoreMesh(core_axis_name="c", subcore_axis_name="s"),
    out_type=(
        jax.ShapeDtypeStruct((BATCH, EMBED), jnp.float32),
        jax.ShapeDtypeStruct((BATCH, EMBED), jnp.float32),
    ),
    scratch_types=[
        pltpu.VMEM((B_PER_W,), jnp.int32),
        pltpu.VMEM((B_PER_W,), jnp.int32),
        pltpu.VMEM((B_PER_W, EMBED), jnp.float32),
        pltpu.VMEM((B_PER_W, EMBED), jnp.float32),
        pltpu.SemaphoreType.DMA,
        pltpu.SemaphoreType.DMA,
    ],
    compiler_params=pltpu.CompilerParams(use_tc_tiling_on_sc=False),
)
def _bpr_gather(
    user_hbm,
    item_hbm,
    user_table_hbm,
    item_table_hbm,
    user_out_hbm,
    item_out_hbm,
    uidx_v,
    iidx_v,
    urows_v,
    irows_v,
    usem,
    isem,
):
    wid = lax.axis_index("s") * NUM_CORES + lax.axis_index("c")
    base = wid * B_PER_W

    pltpu.sync_copy(user_hbm.at[pl.ds(base, B_PER_W)], uidx_v)
    pltpu.sync_copy(item_hbm.at[pl.ds(base, B_PER_W)], iidx_v)

    ucopies = []
    icopies = []
    for j in range(N_CHUNKS):
        sl = pl.ds(j * CHUNK, CHUNK)
        ucopies.append(
            pltpu.async_copy(
                user_table_hbm.at[uidx_v.at[sl]], urows_v.at[sl], usem
            )
        )
        icopies.append(
            pltpu.async_copy(
                item_table_hbm.at[iidx_v.at[sl]], irows_v.at[sl], isem
            )
        )
    for c in ucopies:
        c.wait()
    pltpu.sync_copy(urows_v, user_out_hbm.at[pl.ds(base, B_PER_W)])
    for c in icopies:
        c.wait()
    pltpu.sync_copy(irows_v, item_out_hbm.at[pl.ds(base, B_PER_W)])


def kernel(user, item, user_table, item_table):
    return _bpr_gather(user, item, user_table, item_table)
